# baseline (speedup 1.0000x reference)
import functools
import math

import jax
import jax.numpy as jnp
import numpy as np
from jax import lax
from jax.experimental import pallas as pl
from jax.experimental.pallas import tpu as pltpu
from jax.experimental.pallas import tpu_sc as plsc

VOCAB = 100000
D = 768
B = 4
M = 4096
EPS = 1e-5
L = 16
NV = D // L

NC, NS = 2, 16
NW = NC * NS
MPW = M // NW
MC = 32
NK = MPW // MC
NBUF = 4


def _positionals():
    pos = np.arange(M, dtype=np.float32)[:, None]
    div = np.exp(np.arange(0, D, 2, dtype=np.float32) * (-math.log(10000.0) / D))
    pe = np.zeros((M, D), dtype=np.float32)
    pe[:, 0::2] = np.sin(pos * div)
    pe[:, 1::2] = np.cos(pos * div)
    return pe


_POS = _positionals()


def _interleave_cols(a):
    shp = a.shape
    a = a.reshape(shp[:-1] + (D // 32, 2, 16))
    a = jnp.swapaxes(a, -1, -2)
    return a.reshape(shp)


def _rsqrt(v):
    xi = lax.bitcast_convert_type(v, jnp.int32)
    yi = jnp.int32(0x5F3759DF) - lax.shift_right_arithmetic(xi, 1)
    y = lax.bitcast_convert_type(yi, jnp.float32)
    h = v * 0.5
    y = y * (1.5 - h * y * y)
    y = y * (1.5 - h * y * y)
    return y


def _lanesum(v):
    for sh in (1, 2, 4, 8):
        perm = jnp.arange(L, dtype=jnp.int32) ^ sh
        v = v + v.at[perm].get(mode="promise_in_bounds")
    return v


def _unpack_bf16_pair(w):
    lo = lax.bitcast_convert_type(lax.shift_left(w, 16), jnp.float32)
    hi = lax.bitcast_convert_type(
        lax.bitwise_and(w, jnp.int32(-65536)), jnp.float32)
    return lo, hi


def _compute_chunk(buf, pos_v, gam_v, idx_v, b, k):
    @plsc.parallel_loop(0, MC, unroll=4)
    def row_body(r):
        iv = idx_v[b, pl.ds(k * MC + ((r >> 4) << 4), L)]
        lane = jnp.full((L,), r & 15, dtype=jnp.int32)
        iv_b = iv.at[lane].get(mode="promise_in_bounds")
        ii = jnp.where(iv_b != 0, 1.0, 0.0)
        acc = [jnp.zeros((L,), jnp.float32) for _ in range(4)]
        acc2 = [jnp.zeros((L,), jnp.float32) for _ in range(4)]
        for v in range(NV):
            xv = buf[r, pl.ds(v * L, L)]
            acc[v & 3] = acc[v & 3] + xv
            acc2[v & 3] = acc2[v & 3] + xv * xv
        mean = _lanesum((acc[0] + acc[1]) + (acc[2] + acc[3])) * (1.0 / D)
        var = (_lanesum((acc2[0] + acc2[1]) + (acc2[2] + acc2[3])) * (1.0 / D)
               - mean * mean)
        rstd = _rsqrt(jnp.maximum(var, 0.0) + EPS)
        s = rstd * ii
        for g in range(NV // 2):
            slp = pl.ds(g * L, L)
            pa, pb = _unpack_bf16_pair(pos_v[r, slp])
            ga, gb = _unpack_bf16_pair(gam_v[slp])
            sla = pl.ds(g * 32, L)
            slb = pl.ds(g * 32 + L, L)
            xa = buf[r, sla]
            xb = buf[r, slb]
            buf[r, sla] = (xa - mean) * s * ga + pa
            buf[r, slb] = (xb - mean) * s * gb + pb


def _make_kernel():
    mesh = plsc.VectorSubcoreMesh(core_axis_name="c", subcore_axis_name="s")

    @functools.partial(
        pl.kernel,
        mesh=mesh,
        out_type=jax.ShapeDtypeStruct((B, M, D), jnp.float32),
        scratch_types=[
            pltpu.VMEM((B, MPW), jnp.int32),
            pltpu.VMEM((MC, D // 2), jnp.int32),
            pltpu.VMEM((MC, D), jnp.float32),
            pltpu.VMEM((MC, D), jnp.float32),
            pltpu.VMEM((MC, D), jnp.float32),
            pltpu.VMEM((MC, D), jnp.float32),
            pltpu.VMEM((D // 2,), jnp.int32),
            pltpu.SemaphoreType.DMA,
            pltpu.SemaphoreType.DMA,
        ],
    )
    def pe_kernel(x_hbm, table_hbm, gamma_hbm, pos_hbm, out_hbm,
                  idx_v, pos_v, buf0, buf1, buf2, buf3, gam_v, gsem, wsem):
        bufs = (buf0, buf1, buf2, buf3)
        wid = lax.axis_index("s") * NC + lax.axis_index("c")
        w0 = wid * MPW

        pltpu.sync_copy(gamma_hbm, gam_v)
        for b in range(B):
            pltpu.sync_copy(x_hbm.at[b, pl.ds(w0, MPW)], idx_v.at[b])

        def start_gather(b, k):
            pltpu.async_copy(
                table_hbm.at[idx_v.at[b, pl.ds(k * MC, MC)]], bufs[b], gsem)

        def wait_gather(b, k):
            pltpu.make_async_copy(
                table_hbm.at[idx_v.at[b, pl.ds(k * MC, MC)]], bufs[b], gsem
            ).wait()

        def start_write(b, k):
            pltpu.async_copy(
                bufs[b], out_hbm.at[b, pl.ds(w0 + k * MC, MC)], wsem)

        def wait_write(b, k):
            pltpu.make_async_copy(
                bufs[b], out_hbm.at[b, pl.ds(w0 + k * MC, MC)], wsem
            ).wait()

        start_gather(0, 0)
        start_gather(1, 0)

        def k_body(k, _):
            pltpu.sync_copy(pos_hbm.at[pl.ds(w0 + k * MC, MC)], pos_v)
            for b in range(B):
                wait_gather(b, k)
                _compute_chunk(bufs[b], pos_v, gam_v, idx_v, b, k)
                start_write(b, k)
                b2 = (b + 2) % NBUF
                if b < 2:
                    @pl.when(k > 0)
                    def _():
                        wait_write(b2, k - 1)
                    start_gather(b2, k)
                else:
                    wait_write(b2, k)

                    @pl.when(k < NK - 1)
                    def _():
                        start_gather(b2, k + 1)
            return 0

        lax.fori_loop(0, NK, k_body, 0)
        wait_write(2, NK - 1)
        wait_write(3, NK - 1)

    return pe_kernel


_PE = _make_kernel()


def kernel(x, table, gamma, beta):
    pos2 = _interleave_cols(jnp.asarray(_POS) + beta[None, :])
    pos2 = lax.bitcast_convert_type(
        pos2.astype(jnp.bfloat16).reshape(M, D // 2, 2), jnp.int32)
    gamma2 = lax.bitcast_convert_type(
        _interleave_cols(gamma).astype(jnp.bfloat16).reshape(D // 2, 2),
        jnp.int32)
    return _PE(x, table, gamma2, pos2)


# speedup vs baseline: 1.0416x; 1.0416x over previous
import functools
import math

import jax
import jax.numpy as jnp
import numpy as np
from jax import lax
from jax.experimental import pallas as pl
from jax.experimental.pallas import tpu as pltpu
from jax.experimental.pallas import tpu_sc as plsc

VOCAB = 100000
D = 768
B = 4
M = 4096
EPS = 1e-5
L = 16
NV = D // L

NC, NS = 2, 16
NW = NC * NS
MPW = M // NW
MC = 32
NK = MPW // MC
NBUF = 4


def _positionals():
    pos = np.arange(M, dtype=np.float32)[:, None]
    div = np.exp(np.arange(0, D, 2, dtype=np.float32) * (-math.log(10000.0) / D))
    pe = np.zeros((M, D), dtype=np.float32)
    pe[:, 0::2] = np.sin(pos * div)
    pe[:, 1::2] = np.cos(pos * div)
    return pe


_POS = _positionals()


def _interleave_cols(a):
    shp = a.shape
    a = a.reshape(shp[:-1] + (D // 32, 2, 16))
    a = jnp.swapaxes(a, -1, -2)
    return a.reshape(shp)


def _rsqrt(v):
    xi = lax.bitcast_convert_type(v, jnp.int32)
    yi = jnp.int32(0x5F3759DF) - lax.shift_right_arithmetic(xi, 1)
    y = lax.bitcast_convert_type(yi, jnp.float32)
    h = v * 0.5
    y = y * (1.5 - h * y * y)
    y = y * (1.5 - h * y * y)
    return y


def _lanesum(v):
    for sh in (1, 2, 4, 8):
        perm = jnp.arange(L, dtype=jnp.int32) ^ sh
        v = v + v.at[perm].get(mode="promise_in_bounds")
    return v


def _unpack_bf16_pair(w):
    lo = lax.bitcast_convert_type(lax.shift_left(w, 16), jnp.float32)
    hi = lax.bitcast_convert_type(
        lax.bitwise_and(w, jnp.int32(-65536)), jnp.float32)
    return lo, hi


def _compute_chunk(buf, pos_v, gam_v, idx_v, b, k):
    @plsc.parallel_loop(0, MC, unroll=3)
    def row_body(r):
        iv = idx_v[b, pl.ds(k * MC + ((r >> 4) << 4), L)]
        lane = jnp.full((L,), r & 15, dtype=jnp.int32)
        iv_b = iv.at[lane].get(mode="promise_in_bounds")
        ii = jnp.where(iv_b != 0, 1.0, 0.0)
        acc = [jnp.zeros((L,), jnp.float32) for _ in range(4)]
        acc2 = [jnp.zeros((L,), jnp.float32) for _ in range(4)]
        for v in range(NV):
            xv = buf[r, pl.ds(v * L, L)]
            acc[v & 3] = acc[v & 3] + xv
            acc2[v & 3] = acc2[v & 3] + xv * xv
        mean = _lanesum((acc[0] + acc[1]) + (acc[2] + acc[3])) * (1.0 / D)
        var = (_lanesum((acc2[0] + acc2[1]) + (acc2[2] + acc2[3])) * (1.0 / D)
               - mean * mean)
        rstd = _rsqrt(jnp.maximum(var, 0.0) + EPS)
        s = rstd * ii
        for g in range(NV // 2):
            slp = pl.ds(g * L, L)
            pa, pb = _unpack_bf16_pair(pos_v[r, slp])
            ga, gb = _unpack_bf16_pair(gam_v[slp])
            sla = pl.ds(g * 32, L)
            slb = pl.ds(g * 32 + L, L)
            xa = buf[r, sla]
            xb = buf[r, slb]
            buf[r, sla] = (xa - mean) * s * ga + pa
            buf[r, slb] = (xb - mean) * s * gb + pb


def _make_kernel():
    mesh = plsc.VectorSubcoreMesh(core_axis_name="c", subcore_axis_name="s")

    @functools.partial(
        pl.kernel,
        mesh=mesh,
        out_type=jax.ShapeDtypeStruct((B, M, D), jnp.float32),
        scratch_types=[
            pltpu.VMEM((B, MPW), jnp.int32),
            pltpu.VMEM((MC, D // 2), jnp.int32),
            pltpu.VMEM((MC, D), jnp.float32),
            pltpu.VMEM((MC, D), jnp.float32),
            pltpu.VMEM((MC, D), jnp.float32),
            pltpu.VMEM((MC, D), jnp.float32),
            pltpu.VMEM((D // 2,), jnp.int32),
            pltpu.SemaphoreType.DMA,
            pltpu.SemaphoreType.DMA,
        ],
    )
    def pe_kernel(x_hbm, table_hbm, gamma_hbm, pos_hbm, out_hbm,
                  idx_v, pos_v, buf0, buf1, buf2, buf3, gam_v, gsem, wsem):
        bufs = (buf0, buf1, buf2, buf3)
        wid = lax.axis_index("s") * NC + lax.axis_index("c")
        w0 = wid * MPW

        pltpu.sync_copy(gamma_hbm, gam_v)
        for b in range(B):
            pltpu.sync_copy(x_hbm.at[b, pl.ds(w0, MPW)], idx_v.at[b])

        def start_gather(b, k):
            pltpu.async_copy(
                table_hbm.at[idx_v.at[b, pl.ds(k * MC, MC)]], bufs[b], gsem)

        def wait_gather(b, k):
            pltpu.make_async_copy(
                table_hbm.at[idx_v.at[b, pl.ds(k * MC, MC)]], bufs[b], gsem
            ).wait()

        def start_write(b, k):
            pltpu.async_copy(
                bufs[b], out_hbm.at[b, pl.ds(w0 + k * MC, MC)], wsem)

        def wait_write(b, k):
            pltpu.make_async_copy(
                bufs[b], out_hbm.at[b, pl.ds(w0 + k * MC, MC)], wsem
            ).wait()

        start_gather(0, 0)
        start_gather(1, 0)

        def k_body(k, _):
            pltpu.sync_copy(pos_hbm.at[pl.ds(w0 + k * MC, MC)], pos_v)
            for b in range(B):
                wait_gather(b, k)
                _compute_chunk(bufs[b], pos_v, gam_v, idx_v, b, k)
                start_write(b, k)
                b2 = (b + 2) % NBUF
                if b < 2:
                    @pl.when(k > 0)
                    def _():
                        wait_write(b2, k - 1)
                    start_gather(b2, k)
                else:
                    wait_write(b2, k)

                    @pl.when(k < NK - 1)
                    def _():
                        start_gather(b2, k + 1)
            return 0

        lax.fori_loop(0, NK, k_body, 0)
        wait_write(2, NK - 1)
        wait_write(3, NK - 1)

    return pe_kernel


_PE = _make_kernel()


def kernel(x, table, gamma, beta):
    pos2 = _interleave_cols(jnp.asarray(_POS) + beta[None, :])
    pos2 = lax.bitcast_convert_type(
        pos2.astype(jnp.bfloat16).reshape(M, D // 2, 2), jnp.int32)
    gamma2 = lax.bitcast_convert_type(
        _interleave_cols(gamma).astype(jnp.bfloat16).reshape(D // 2, 2),
        jnp.int32)
    return _PE(x, table, gamma2, pos2)


# speedup vs baseline: 1.0513x; 1.0093x over previous
import functools
import math

import jax
import jax.numpy as jnp
import numpy as np
from jax import lax
from jax.experimental import pallas as pl
from jax.experimental.pallas import tpu as pltpu
from jax.experimental.pallas import tpu_sc as plsc

VOCAB = 100000
D = 768
B = 4
M = 4096
EPS = 1e-5
L = 16
NV = D // L

NC, NS = 2, 16
NW = NC * NS
MPW = M // NW
MC = 32
NK = MPW // MC
NBUF = 4


def _positionals():
    pos = np.arange(M, dtype=np.float32)[:, None]
    div = np.exp(np.arange(0, D, 2, dtype=np.float32) * (-math.log(10000.0) / D))
    pe = np.zeros((M, D), dtype=np.float32)
    pe[:, 0::2] = np.sin(pos * div)
    pe[:, 1::2] = np.cos(pos * div)
    return pe


_POS = _positionals()


def _interleave_cols(a):
    shp = a.shape
    a = a.reshape(shp[:-1] + (D // 32, 2, 16))
    a = jnp.swapaxes(a, -1, -2)
    return a.reshape(shp)


def _rsqrt(v):
    xi = lax.bitcast_convert_type(v, jnp.int32)
    yi = jnp.int32(0x5F3759DF) - lax.shift_right_arithmetic(xi, 1)
    y = lax.bitcast_convert_type(yi, jnp.float32)
    h = v * 0.5
    y = y * (1.5 - h * y * y)
    y = y * (1.5 - h * y * y)
    return y


def _lanesum(v):
    for sh in (1, 2, 4, 8):
        perm = jnp.arange(L, dtype=jnp.int32) ^ sh
        v = v + v.at[perm].get(mode="promise_in_bounds")
    return v


def _unpack_bf16_pair(w):
    lo = lax.bitcast_convert_type(lax.shift_left(w, 16), jnp.float32)
    hi = lax.bitcast_convert_type(
        lax.bitwise_and(w, jnp.int32(-65536)), jnp.float32)
    return lo, hi


def _compute_chunk(buf, pos_v, gam_v, idx_v, b, k):
    @plsc.parallel_loop(0, MC, unroll=2)
    def row_body(r):
        iv = idx_v[b, pl.ds(k * MC + ((r >> 4) << 4), L)]
        lane = jnp.full((L,), r & 15, dtype=jnp.int32)
        iv_b = iv.at[lane].get(mode="promise_in_bounds")
        ii = jnp.where(iv_b != 0, 1.0, 0.0)
        acc = [jnp.zeros((L,), jnp.float32) for _ in range(4)]
        acc2 = [jnp.zeros((L,), jnp.float32) for _ in range(4)]
        for v in range(NV):
            xv = buf[r, pl.ds(v * L, L)]
            acc[v & 3] = acc[v & 3] + xv
            acc2[v & 3] = acc2[v & 3] + xv * xv
        mean = _lanesum((acc[0] + acc[1]) + (acc[2] + acc[3])) * (1.0 / D)
        var = (_lanesum((acc2[0] + acc2[1]) + (acc2[2] + acc2[3])) * (1.0 / D)
               - mean * mean)
        rstd = _rsqrt(jnp.maximum(var, 0.0) + EPS)
        s = rstd * ii
        for g in range(NV // 2):
            slp = pl.ds(g * L, L)
            pa, pb = _unpack_bf16_pair(pos_v[r, slp])
            ga, gb = _unpack_bf16_pair(gam_v[slp])
            sla = pl.ds(g * 32, L)
            slb = pl.ds(g * 32 + L, L)
            xa = buf[r, sla]
            xb = buf[r, slb]
            buf[r, sla] = (xa - mean) * s * ga + pa
            buf[r, slb] = (xb - mean) * s * gb + pb


def _make_kernel():
    mesh = plsc.VectorSubcoreMesh(core_axis_name="c", subcore_axis_name="s")

    @functools.partial(
        pl.kernel,
        mesh=mesh,
        out_type=jax.ShapeDtypeStruct((B, M, D), jnp.float32),
        scratch_types=[
            pltpu.VMEM((B, MPW), jnp.int32),
            pltpu.VMEM((MC, D // 2), jnp.int32),
            pltpu.VMEM((MC, D), jnp.float32),
            pltpu.VMEM((MC, D), jnp.float32),
            pltpu.VMEM((MC, D), jnp.float32),
            pltpu.VMEM((MC, D), jnp.float32),
            pltpu.VMEM((D // 2,), jnp.int32),
            pltpu.SemaphoreType.DMA,
            pltpu.SemaphoreType.DMA,
        ],
    )
    def pe_kernel(x_hbm, table_hbm, gamma_hbm, pos_hbm, out_hbm,
                  idx_v, pos_v, buf0, buf1, buf2, buf3, gam_v, gsem, wsem):
        bufs = (buf0, buf1, buf2, buf3)
        wid = lax.axis_index("s") * NC + lax.axis_index("c")
        w0 = wid * MPW

        pltpu.sync_copy(gamma_hbm, gam_v)
        for b in range(B):
            pltpu.sync_copy(x_hbm.at[b, pl.ds(w0, MPW)], idx_v.at[b])

        def start_gather(b, k):
            pltpu.async_copy(
                table_hbm.at[idx_v.at[b, pl.ds(k * MC, MC)]], bufs[b], gsem)

        def wait_gather(b, k):
            pltpu.make_async_copy(
                table_hbm.at[idx_v.at[b, pl.ds(k * MC, MC)]], bufs[b], gsem
            ).wait()

        def start_write(b, k):
            pltpu.async_copy(
                bufs[b], out_hbm.at[b, pl.ds(w0 + k * MC, MC)], wsem)

        def wait_write(b, k):
            pltpu.make_async_copy(
                bufs[b], out_hbm.at[b, pl.ds(w0 + k * MC, MC)], wsem
            ).wait()

        start_gather(0, 0)
        start_gather(1, 0)

        def k_body(k, _):
            pltpu.sync_copy(pos_hbm.at[pl.ds(w0 + k * MC, MC)], pos_v)
            for b in range(B):
                wait_gather(b, k)
                _compute_chunk(bufs[b], pos_v, gam_v, idx_v, b, k)
                start_write(b, k)
                b2 = (b + 2) % NBUF
                if b < 2:
                    @pl.when(k > 0)
                    def _():
                        wait_write(b2, k - 1)
                    start_gather(b2, k)
                else:
                    wait_write(b2, k)

                    @pl.when(k < NK - 1)
                    def _():
                        start_gather(b2, k + 1)
            return 0

        lax.fori_loop(0, NK, k_body, 0)
        wait_write(2, NK - 1)
        wait_write(3, NK - 1)

    return pe_kernel


_PE = _make_kernel()


def kernel(x, table, gamma, beta):
    pos2 = _interleave_cols(jnp.asarray(_POS) + beta[None, :])
    pos2 = lax.bitcast_convert_type(
        pos2.astype(jnp.bfloat16).reshape(M, D // 2, 2), jnp.int32)
    gamma2 = lax.bitcast_convert_type(
        _interleave_cols(gamma).astype(jnp.bfloat16).reshape(D // 2, 2),
        jnp.int32)
    return _PE(x, table, gamma2, pos2)
